# baseline (speedup 1.0000x reference)
import functools

import jax
import jax.numpy as jnp
from jax import lax
from jax.experimental import pallas as pl
from jax.experimental.pallas import tpu as pltpu
from jax.experimental.pallas import tpu_sc as plsc

BATCH = 16384
FACTOR = 32
NC = 2
NS = 16
NW = NC * NS
BPW = BATCH // NW
CHUNK = 128
NCHUNK = BPW // CHUNK

_mesh = plsc.VectorSubcoreMesh(core_axis_name="c", subcore_axis_name="s")


@functools.partial(
    pl.kernel,
    out_type=jax.ShapeDtypeStruct((BATCH,), jnp.float32),
    mesh=_mesh,
    compiler_params=pltpu.CompilerParams(
        needs_layout_passes=False, use_tc_tiling_on_sc=False),
    scratch_types=[
        pltpu.VMEM((BPW,), jnp.int32),
        pltpu.VMEM((BPW,), jnp.int32),
        pltpu.VMEM((BPW, FACTOR), jnp.float32),
        pltpu.VMEM((BPW, FACTOR), jnp.float32),
        pltpu.VMEM((BPW,), jnp.float32),
        pltpu.SemaphoreType.DMA,
    ],
)
def _mf_kernel(user_hbm, item_hbm, uf_hbm, if_hbm, out_hbm,
               uidx_v, iidx_v, urows_v, irows_v, out_v, sem):
    wid = lax.axis_index("s") * NC + lax.axis_index("c")
    base = wid * BPW

    pltpu.sync_copy(user_hbm.at[pl.ds(base, BPW)], uidx_v)
    pltpu.sync_copy(item_hbm.at[pl.ds(base, BPW)], iidx_v)

    copies = []
    for c in range(NCHUNK):
        copies.append(pltpu.async_copy(
            uf_hbm.at[uidx_v.at[pl.ds(c * CHUNK, CHUNK)]],
            urows_v.at[pl.ds(c * CHUNK, CHUNK), :], sem))
        copies.append(pltpu.async_copy(
            if_hbm.at[iidx_v.at[pl.ds(c * CHUNK, CHUNK)]],
            irows_v.at[pl.ds(c * CHUNK, CHUNK), :], sem))
    for cp in copies:
        cp.wait()

    lane = lax.iota(jnp.int32, 16)

    def group_body(g, _):
        rows = g * 16 + lane
        acc = jnp.zeros((16,), jnp.float32)
        for f in range(FACTOR):
            fs = jnp.full((16,), f, jnp.int32)
            u = plsc.load_gather(urows_v, [rows, fs])
            v = plsc.load_gather(irows_v, [rows, fs])
            acc = acc + u * v
        out_v[pl.ds(g * 16, 16)] = acc
        return 0

    lax.fori_loop(0, BPW // 16, group_body, 0)

    pltpu.sync_copy(out_v, out_hbm.at[pl.ds(base, BPW)])


def kernel(user, item, user_factors, item_factors):
    return _mf_kernel(user, item, user_factors, item_factors)


# speedup vs baseline: 4.3768x; 4.3768x over previous
import functools

import jax
import jax.numpy as jnp
from jax import lax
from jax.experimental import pallas as pl
from jax.experimental.pallas import tpu as pltpu
from jax.experimental.pallas import tpu_sc as plsc

BATCH = 16384
FACTOR = 32
NC = 2
NS = 16
NW = NC * NS
BPW = BATCH // NW
RING = 8
NGRP = BPW // 16

_mesh = plsc.VectorSubcoreMesh(core_axis_name="c", subcore_axis_name="s")


@functools.partial(
    pl.kernel,
    out_type=jax.ShapeDtypeStruct((BATCH,), jnp.float32),
    mesh=_mesh,
    compiler_params=pltpu.CompilerParams(
        needs_layout_passes=False, use_tc_tiling_on_sc=True),
    scratch_types=[
        pltpu.VMEM((BPW,), jnp.int32),
        pltpu.VMEM((BPW,), jnp.int32),
        pltpu.VMEM((FACTOR, RING * 256), jnp.float32),
        pltpu.VMEM((BPW,), jnp.float32),
    ] + [pltpu.SemaphoreType.DMA] * RING,
)
def _mf_kernel(user_hbm, item_hbm, uft_hbm, ift_hbm, out_hbm,
               uidx_v, iidx_v, cols_v, out_v, *sems):
    wid = lax.axis_index("s") * NC + lax.axis_index("c")
    base = wid * BPW

    pltpu.sync_copy(user_hbm.at[pl.ds(base, BPW)], uidx_v)
    pltpu.sync_copy(item_hbm.at[pl.ds(base, BPW)], iidx_v)

    lane = lax.iota(jnp.int32, 16)

    def fire(slot, iu, iv):
        cu = pl.multiple_of((iu >> 7) << 7, 128)
        ci = pl.multiple_of((iv >> 7) << 7, 128)
        pltpu.async_copy(uft_hbm.at[:, pl.ds(cu, 128)],
                         cols_v.at[:, pl.ds(slot * 256, 128)], sems[slot])
        pltpu.async_copy(ift_hbm.at[:, pl.ds(ci, 128)],
                         cols_v.at[:, pl.ds(slot * 256 + 128, 128)],
                         sems[slot])

    def wait_slot(slot):
        for _ in range(2):
            pltpu.make_async_copy(
                uft_hbm.at[:, pl.ds(0, 128)],
                cols_v.at[:, pl.ds(slot * 256, 128)], sems[slot]).wait()

    uvec0 = uidx_v[pl.ds(0, 16)]
    ivec0 = iidx_v[pl.ds(0, 16)]
    for j in range(RING):
        fire(j, uvec0[j], ivec0[j])

    def group_body(g, _):
        uvec = uidx_v[pl.ds(g * 16, 16)]
        ivec = iidx_v[pl.ds(g * 16, 16)]
        nxt = lax.rem((g + 1) * 16, BPW)
        nuvec = uidx_v[pl.ds(nxt, 16)]
        nivec = iidx_v[pl.ds(nxt, 16)]
        acc = jnp.zeros((16,), jnp.float32)
        for j in range(16):
            slot = j % RING
            wait_slot(slot)
            ucol = jnp.full((16,), slot * 256, jnp.int32) + (uvec[j] & 127)
            icol = jnp.full((16,), slot * 256 + 128, jnp.int32) + (ivec[j] & 127)
            u0 = plsc.load_gather(cols_v, [lane, ucol])
            u1 = plsc.load_gather(cols_v, [lane + 16, ucol])
            v0 = plsc.load_gather(cols_v, [lane, icol])
            v1 = plsc.load_gather(cols_v, [lane + 16, icol])
            p = u0 * v0 + u1 * v1
            s = jnp.sum(p)
            acc = jnp.where(lane == j, s, acc)
            if j < RING:
                @pl.when(g * 16 + j + RING < BPW)
                def _():
                    fire(slot, uvec[j + RING], ivec[j + RING])
            else:
                @pl.when((g + 1) * 16 + (j - RING) < BPW)
                def _():
                    fire(slot, nuvec[j - RING], nivec[j - RING])
        out_v[pl.ds(g * 16, 16)] = acc
        return 0

    lax.fori_loop(0, NGRP, group_body, 0)

    pltpu.sync_copy(out_v, out_hbm.at[pl.ds(base, BPW)])


def kernel(user, item, user_factors, item_factors):
    return _mf_kernel(user, item, user_factors.T, item_factors.T)
